# baseline (speedup 1.0000x reference)
import functools

import jax
import jax.numpy as jnp
from jax import lax
from jax.experimental import pallas as pl
from jax.experimental.pallas import tpu as pltpu
from jax.experimental.pallas import tpu_sc as plsc

N_NODES = 10000
N_EDGES = 640000
NUM_GRAPHS = 128

NP = 10016
D = 128
DX = 48
DAUG = 144
NW = 32
NTILE = 16

ECHUNK = 64
EIB = 40
ENB0 = 4
ENB1 = 12
ENB = max(ENB0, ENB1)
EPAD0 = NTILE * ENB0 * EIB * ECHUNK
EPAD1 = NTILE * ENB1 * EIB * ECHUNK
NBUF = 5

PCHUNK = 64
PIB = 5
PPAD = NW * PIB * PCHUNK
NPOOL = 144

BM = 2504


def _make_sc_scatter(d, nb, ib, chunk, nout, nbuf=NBUF):
    rows_per_tile = nout // NTILE
    assert ib % nbuf == 0
    mesh = plsc.VectorSubcoreMesh(core_axis_name="c", subcore_axis_name="s")

    @functools.partial(
        pl.kernel,
        out_type=jax.ShapeDtypeStruct((2, nout, d), jnp.float32),
        mesh=mesh,
        scratch_types=(
            [pltpu.VMEM((ib, chunk), jnp.int32),
             pltpu.VMEM((ib, chunk), jnp.int32)]
            + [pltpu.VMEM((chunk, d), jnp.float32) for _ in range(nbuf)]
            + [pltpu.SemaphoreType.DMA for _ in range(2 * nbuf)]
            + [pltpu.VMEM_SHARED((nout, d), jnp.float32)]
        ),
        compiler_params=pltpu.CompilerParams(use_tc_tiling_on_sc=False),
    )
    def scatter_k(table, srcs, dsts, zeros, out, src_v, dst_v, *rest):
        rows = rest[:nbuf]
        sg = rest[nbuf:2 * nbuf]
        ss = rest[2 * nbuf:3 * nbuf]
        acc = rest[3 * nbuf]
        cid = lax.axis_index("c")
        sid = lax.axis_index("s")
        wid = cid * NTILE + sid
        pltpu.sync_copy(
            zeros.at[pl.ds(sid * rows_per_tile, rows_per_tile)],
            acc.at[pl.ds(sid * rows_per_tile, rows_per_tile)],
        )
        plsc.subcore_barrier()

        def outer(ob, carry):
            pltpu.sync_copy(srcs.at[wid, ob], src_v)
            pltpu.sync_copy(dsts.at[wid, ob], dst_v)

            def body(jj, c2):
                base = jj * nbuf
                gds = [pltpu.async_copy(table.at[src_v.at[base + k]],
                                        rows[k], sg[k])
                       for k in range(nbuf)]
                sds = []
                for k in range(nbuf):
                    gds[k].wait()
                    sds.append(pltpu.async_copy(
                        rows[k], acc.at[dst_v.at[base + k]], ss[k], add=True))
                for k in range(nbuf):
                    sds[k].wait()
                return c2

            lax.fori_loop(0, ib // nbuf, body, carry)
            return carry

        if isinstance(nb, tuple):
            nb_eff = jnp.where(cid == 0, nb[0], nb[1])
        else:
            nb_eff = nb
        lax.fori_loop(0, nb_eff, outer, 0)
        plsc.subcore_barrier()
        pltpu.sync_copy(
            acc.at[pl.ds(sid * rows_per_tile, rows_per_tile)],
            out.at[cid, pl.ds(sid * rows_per_tile, rows_per_tile)],
        )

    return scatter_k


_sc_scatter_x = _make_sc_scatter(DX, (ENB0, ENB1), EIB, ECHUNK, NP)
_sc_scatter_h = _make_sc_scatter(D, (ENB0, ENB1), EIB, ECHUNK, NP)
_sc_pool = _make_sc_scatter(DAUG, 1, PIB, PCHUNK, NPOOL)

def _ones_col_block(bm):
    col = lax.broadcasted_iota(jnp.int32, (bm, DAUG - D), 1)
    return jnp.where(col == 0, 1.0, 0.0).astype(jnp.float32)


_WSPEC = pl.BlockSpec((D, D), lambda i: (0, 0))
_BSPEC = pl.BlockSpec((1, D), lambda i: (0, 0))
_MSPEC = pl.BlockSpec((BM, D), lambda i: (i, 0))
_DEGSPEC = pl.BlockSpec((BM, 1), lambda i: (i, 0))


def _tc_layer0(parts, x_aug, w1aug, w2aug, w3aug):
    def body(p_ref, x_ref, w1_ref, w2_ref, w3_ref, h_ref, deg_ref):
        hx = p_ref[0] + p_ref[1]
        xb = x_ref[...]
        deg = hx[:, 37:38]
        agg = jnp.dot(hx, w1_ref[...], preferred_element_type=jnp.float32)
        bb = jnp.dot(xb, w2_ref[...], preferred_element_type=jnp.float32)
        cc = jnp.dot(xb, w3_ref[...], preferred_element_type=jnp.float32)
        h_ref[...] = jnp.maximum(agg - deg * bb + cc, 0.0)
        deg_ref[...] = deg

    xwspec = pl.BlockSpec((DX, D), lambda i: (0, 0))
    return pl.pallas_call(
        body,
        grid=(NP // BM,),
        in_specs=[pl.BlockSpec((2, BM, DX), lambda i: (0, i, 0)),
                  pl.BlockSpec((BM, DX), lambda i: (i, 0)),
                  xwspec, xwspec, xwspec],
        out_specs=[_MSPEC, _DEGSPEC],
        out_shape=[jax.ShapeDtypeStruct((NP, D), jnp.float32),
                   jax.ShapeDtypeStruct((NP, 1), jnp.float32)],
    )(parts, x_aug, w1aug, w2aug, w3aug)


def _tc_layer(parts, h, deg, w1, b1, w2, w3, b3, aug_out):
    def body(p_ref, h_ref, deg_ref, w1_ref, b1_ref, w2_ref, w3_ref, b3_ref,
             out_ref):
        hagg = p_ref[0] + p_ref[1]
        hb = h_ref[...]
        deg = deg_ref[...]
        agg = jnp.dot(hagg, w1_ref[...], preferred_element_type=jnp.float32)
        bb = jnp.dot(hb, w2_ref[...], preferred_element_type=jnp.float32)
        cc = jnp.dot(hb, w3_ref[...], preferred_element_type=jnp.float32)
        hn = jnp.maximum(agg + deg * (b1_ref[...] - bb) + cc + b3_ref[...],
                         0.0)
        if aug_out:
            out_ref[:, :D] = hn
            out_ref[:, D:] = _ones_col_block(BM)
        else:
            out_ref[...] = hn

    dout = DAUG if aug_out else D
    return pl.pallas_call(
        body,
        grid=(NP // BM,),
        in_specs=[pl.BlockSpec((2, BM, D), lambda i: (0, i, 0)),
                  _MSPEC, _DEGSPEC, _WSPEC, _BSPEC, _WSPEC, _WSPEC, _BSPEC],
        out_specs=pl.BlockSpec((BM, dout), lambda i: (i, 0)),
        out_shape=jax.ShapeDtypeStruct((NP, dout), jnp.float32),
    )(parts, h, deg, w1, b1, w2, w3, b3)


def _tc_head(pool_parts, wf1, bf1, wf2, bf2):
    def body(p_ref, wf1_ref, bf1_ref, wf2_ref, bf2_ref, out_ref):
        p0 = p_ref[0]
        p1 = p_ref[1]
        sums = p0[:NUM_GRAPHS, :D] + p1[:NUM_GRAPHS, :D]
        cnt = p0[:NUM_GRAPHS, D:D + 1] + p1[:NUM_GRAPHS, D:D + 1]
        gx = sums / jnp.maximum(cnt, 1.0)
        hidden = jnp.maximum(
            jnp.dot(gx, wf1_ref[...], preferred_element_type=jnp.float32)
            + bf1_ref[...], 0.0)
        out_ref[...] = jnp.dot(hidden, wf2_ref[...],
                               preferred_element_type=jnp.float32) + bf2_ref[...]

    return pl.pallas_call(
        body,
        in_specs=[pl.BlockSpec((2, NPOOL, DAUG), lambda: (0, 0, 0)),
                  pl.BlockSpec((D, D), lambda: (0, 0)),
                  pl.BlockSpec((1, D), lambda: (0, 0)),
                  pl.BlockSpec((D, D), lambda: (0, 0)),
                  pl.BlockSpec((1, D), lambda: (0, 0))],
        out_specs=pl.BlockSpec((NUM_GRAPHS, D), lambda: (0, 0)),
        out_shape=jax.ShapeDtypeStruct((NUM_GRAPHS, D), jnp.float32),
    )(pool_parts, wf1, bf1, wf2, bf2)


def kernel(x, edge_index, batch, W1_0, b1_0, W2_0, W3_0, b3_0, W1_1, b1_1,
           W2_1, W3_1, b3_1, W1_2, b1_2, W2_2, W3_2, b3_2, Wf1, bf1, Wf2, bf2):
    f32 = jnp.float32
    nfeat = x.shape[1]
    x_aug = jnp.concatenate(
        [x, jnp.ones((N_NODES, 1), f32),
         jnp.zeros((N_NODES, DX - nfeat - 1), f32)], axis=1)
    x_aug = jnp.pad(x_aug, ((0, NP - N_NODES), (0, 0)))
    zrows = jnp.zeros((DX - nfeat - 1, D), f32)
    w1aug = jnp.concatenate([W1_0, b1_0[None, :], zrows], axis=0)
    w2aug = jnp.concatenate([W2_0, jnp.zeros((1, D), f32), zrows], axis=0)
    w3aug = jnp.concatenate([W3_0, b3_0[None, :], zrows], axis=0)
    wf2 = jnp.pad(Wf2, ((0, 0), (0, D - Wf2.shape[1])))
    bf2p = jnp.pad(bf2, (0, D - bf2.shape[0])).reshape(1, D)
    b1_1r, b3_1r = b1_1.reshape(1, D), b3_1.reshape(1, D)
    b1_2r, b3_2r = b1_2.reshape(1, D), b3_2.reshape(1, D)
    bf1r = bf1.reshape(1, D)

    def _split_idx(flat, pad_val):
        c0 = flat[:EPAD0].reshape(NTILE, ENB0, EIB, ECHUNK)
        c0 = jnp.pad(c0, ((0, 0), (0, ENB - ENB0), (0, 0), (0, 0)),
                     constant_values=pad_val)
        c1 = jnp.concatenate(
            [flat[EPAD0:],
             jnp.full((EPAD0 + EPAD1 - N_EDGES,), pad_val, jnp.int32)]
        ).reshape(NTILE, ENB1, EIB, ECHUNK)
        c1 = jnp.pad(c1, ((0, 0), (0, ENB - ENB1), (0, 0), (0, 0)),
                     constant_values=pad_val)
        return jnp.concatenate([c0, c1], axis=0)

    src = _split_idx(edge_index[0], 0)
    dst = _split_idx(edge_index[1], N_NODES)

    pool_src = jnp.concatenate(
        [jnp.arange(N_NODES, dtype=jnp.int32),
         jnp.zeros((PPAD - N_NODES,), jnp.int32)]
    ).reshape(NW, 1, PIB, PCHUNK)
    pool_dst = jnp.concatenate(
        [batch, jnp.full((PPAD - N_NODES,), NUM_GRAPHS, jnp.int32)]
    ).reshape(NW, 1, PIB, PCHUNK)

    z_x = jnp.zeros((NP, DX), f32)
    z_h = jnp.zeros((NP, D), f32)
    z_pool = jnp.zeros((NPOOL, DAUG), f32)

    parts0 = _sc_scatter_x(x_aug, src, dst, z_x)
    h1, deg = _tc_layer0(parts0, x_aug, w1aug, w2aug, w3aug)
    parts1 = _sc_scatter_h(h1, src, dst, z_h)
    h2 = _tc_layer(parts1, h1, deg, W1_1, b1_1r, W2_1, W3_1, b3_1r, False)
    parts2 = _sc_scatter_h(h2, src, dst, z_h)
    h3_aug = _tc_layer(parts2, h2, deg, W1_2, b1_2r, W2_2, W3_2, b3_2r, True)
    pool_parts = _sc_pool(h3_aug, pool_src, pool_dst, z_pool)
    pred_pad = _tc_head(pool_parts, Wf1, bf1r, wf2, bf2p)
    return pred_pad[:, :Wf2.shape[1]]


# speedup vs baseline: 1.2817x; 1.2817x over previous
import functools

import jax
import jax.numpy as jnp
from jax import lax
from jax.experimental import pallas as pl
from jax.experimental.pallas import tpu as pltpu
from jax.experimental.pallas import tpu_sc as plsc

N_NODES = 10000
N_EDGES = 640000
NUM_GRAPHS = 128

NP = 10016
D = 128
DX = 48
DAUG = 144
NW = 32
NTILE = 16

ECHUNK = 64
EIB = 40
ENB = 8
EPAD = NW * ENB * EIB * ECHUNK
NBUF = 5

PCHUNK = 64
PIB = 5
PPAD = NW * PIB * PCHUNK
NPOOL = 144

BM = 2504


def _make_sc_scatter(d, nb, ib, chunk, nout, nbuf=NBUF):
    rows_per_tile = nout // NTILE
    assert ib % nbuf == 0
    mesh = plsc.VectorSubcoreMesh(core_axis_name="c", subcore_axis_name="s")

    @functools.partial(
        pl.kernel,
        out_type=jax.ShapeDtypeStruct((2, nout, d), jnp.float32),
        mesh=mesh,
        scratch_types=(
            [pltpu.VMEM((ib, chunk), jnp.int32),
             pltpu.VMEM((ib, chunk), jnp.int32)]
            + [pltpu.VMEM((chunk, d), jnp.float32) for _ in range(nbuf)]
            + [pltpu.SemaphoreType.DMA for _ in range(2 * nbuf)]
            + [pltpu.VMEM_SHARED((nout, d), jnp.float32)]
        ),
        compiler_params=pltpu.CompilerParams(use_tc_tiling_on_sc=False),
    )
    def scatter_k(table, srcs, dsts, zeros, out, src_v, dst_v, *rest):
        rows = rest[:nbuf]
        sg = rest[nbuf:2 * nbuf]
        ss = rest[2 * nbuf:3 * nbuf]
        acc = rest[3 * nbuf]
        cid = lax.axis_index("c")
        sid = lax.axis_index("s")
        wid = cid * NTILE + sid
        pltpu.sync_copy(
            zeros.at[pl.ds(sid * rows_per_tile, rows_per_tile)],
            acc.at[pl.ds(sid * rows_per_tile, rows_per_tile)],
        )
        plsc.subcore_barrier()

        def outer(ob, carry):
            pltpu.sync_copy(srcs.at[wid, ob], src_v)
            pltpu.sync_copy(dsts.at[wid, ob], dst_v)

            def body(jj, c2):
                base = jj * nbuf
                gds = [pltpu.async_copy(table.at[src_v.at[base + k]],
                                        rows[k], sg[k])
                       for k in range(nbuf)]
                sds = []
                for k in range(nbuf):
                    gds[k].wait()
                    sds.append(pltpu.async_copy(
                        rows[k], acc.at[dst_v.at[base + k]], ss[k], add=True))
                for k in range(nbuf):
                    sds[k].wait()
                return c2

            lax.fori_loop(0, ib // nbuf, body, carry)
            return carry

        lax.fori_loop(0, nb, outer, 0)
        plsc.subcore_barrier()
        pltpu.sync_copy(
            acc.at[pl.ds(sid * rows_per_tile, rows_per_tile)],
            out.at[cid, pl.ds(sid * rows_per_tile, rows_per_tile)],
        )

    return scatter_k


_sc_scatter_x = _make_sc_scatter(DX, ENB, EIB, ECHUNK, NP)
_sc_scatter_h = _make_sc_scatter(D, ENB, EIB, ECHUNK, NP)
_sc_pool = _make_sc_scatter(DAUG, 1, PIB, PCHUNK, NPOOL)

def _ones_col_block(bm):
    col = lax.broadcasted_iota(jnp.int32, (bm, DAUG - D), 1)
    return jnp.where(col == 0, 1.0, 0.0).astype(jnp.float32)


_WSPEC = pl.BlockSpec((D, D), lambda i: (0, 0))
_BSPEC = pl.BlockSpec((1, D), lambda i: (0, 0))
_MSPEC = pl.BlockSpec((BM, D), lambda i: (i, 0))
_DEGSPEC = pl.BlockSpec((BM, 1), lambda i: (i, 0))


def _tc_layer0(parts, x_aug, w1aug, w2aug, w3aug):
    def body(p_ref, x_ref, w1_ref, w2_ref, w3_ref, h_ref, deg_ref):
        hx = p_ref[0] + p_ref[1]
        xb = x_ref[...]
        deg = hx[:, 37:38]
        agg = jnp.dot(hx, w1_ref[...], preferred_element_type=jnp.float32)
        bb = jnp.dot(xb, w2_ref[...], preferred_element_type=jnp.float32)
        cc = jnp.dot(xb, w3_ref[...], preferred_element_type=jnp.float32)
        h_ref[...] = jnp.maximum(agg - deg * bb + cc, 0.0)
        deg_ref[...] = deg

    xwspec = pl.BlockSpec((DX, D), lambda i: (0, 0))
    return pl.pallas_call(
        body,
        grid=(NP // BM,),
        in_specs=[pl.BlockSpec((2, BM, DX), lambda i: (0, i, 0)),
                  pl.BlockSpec((BM, DX), lambda i: (i, 0)),
                  xwspec, xwspec, xwspec],
        out_specs=[_MSPEC, _DEGSPEC],
        out_shape=[jax.ShapeDtypeStruct((NP, D), jnp.float32),
                   jax.ShapeDtypeStruct((NP, 1), jnp.float32)],
    )(parts, x_aug, w1aug, w2aug, w3aug)


def _tc_layer(parts, h, deg, w1, b1, w2, w3, b3, aug_out):
    def body(p_ref, h_ref, deg_ref, w1_ref, b1_ref, w2_ref, w3_ref, b3_ref,
             out_ref):
        hagg = p_ref[0] + p_ref[1]
        hb = h_ref[...]
        deg = deg_ref[...]
        agg = jnp.dot(hagg, w1_ref[...], preferred_element_type=jnp.float32)
        bb = jnp.dot(hb, w2_ref[...], preferred_element_type=jnp.float32)
        cc = jnp.dot(hb, w3_ref[...], preferred_element_type=jnp.float32)
        hn = jnp.maximum(agg + deg * (b1_ref[...] - bb) + cc + b3_ref[...],
                         0.0)
        if aug_out:
            out_ref[:, :D] = hn
            out_ref[:, D:] = _ones_col_block(BM)
        else:
            out_ref[...] = hn

    dout = DAUG if aug_out else D
    return pl.pallas_call(
        body,
        grid=(NP // BM,),
        in_specs=[pl.BlockSpec((2, BM, D), lambda i: (0, i, 0)),
                  _MSPEC, _DEGSPEC, _WSPEC, _BSPEC, _WSPEC, _WSPEC, _BSPEC],
        out_specs=pl.BlockSpec((BM, dout), lambda i: (i, 0)),
        out_shape=jax.ShapeDtypeStruct((NP, dout), jnp.float32),
    )(parts, h, deg, w1, b1, w2, w3, b3)


def _tc_head(pool_parts, wf1, bf1, wf2, bf2):
    def body(p_ref, wf1_ref, bf1_ref, wf2_ref, bf2_ref, out_ref):
        p0 = p_ref[0]
        p1 = p_ref[1]
        sums = p0[:NUM_GRAPHS, :D] + p1[:NUM_GRAPHS, :D]
        cnt = p0[:NUM_GRAPHS, D:D + 1] + p1[:NUM_GRAPHS, D:D + 1]
        gx = sums / jnp.maximum(cnt, 1.0)
        hidden = jnp.maximum(
            jnp.dot(gx, wf1_ref[...], preferred_element_type=jnp.float32)
            + bf1_ref[...], 0.0)
        out_ref[...] = jnp.dot(hidden, wf2_ref[...],
                               preferred_element_type=jnp.float32) + bf2_ref[...]

    return pl.pallas_call(
        body,
        in_specs=[pl.BlockSpec((2, NPOOL, DAUG), lambda: (0, 0, 0)),
                  pl.BlockSpec((D, D), lambda: (0, 0)),
                  pl.BlockSpec((1, D), lambda: (0, 0)),
                  pl.BlockSpec((D, D), lambda: (0, 0)),
                  pl.BlockSpec((1, D), lambda: (0, 0))],
        out_specs=pl.BlockSpec((NUM_GRAPHS, D), lambda: (0, 0)),
        out_shape=jax.ShapeDtypeStruct((NUM_GRAPHS, D), jnp.float32),
    )(pool_parts, wf1, bf1, wf2, bf2)


def kernel(x, edge_index, batch, W1_0, b1_0, W2_0, W3_0, b3_0, W1_1, b1_1,
           W2_1, W3_1, b3_1, W1_2, b1_2, W2_2, W3_2, b3_2, Wf1, bf1, Wf2, bf2):
    f32 = jnp.float32
    nfeat = x.shape[1]
    x_aug = jnp.concatenate(
        [x, jnp.ones((N_NODES, 1), f32),
         jnp.zeros((N_NODES, DX - nfeat - 1), f32)], axis=1)
    x_aug = jnp.pad(x_aug, ((0, NP - N_NODES), (0, 0)))
    zrows = jnp.zeros((DX - nfeat - 1, D), f32)
    w1aug = jnp.concatenate([W1_0, b1_0[None, :], zrows], axis=0)
    w2aug = jnp.concatenate([W2_0, jnp.zeros((1, D), f32), zrows], axis=0)
    w3aug = jnp.concatenate([W3_0, b3_0[None, :], zrows], axis=0)
    wf2 = jnp.pad(Wf2, ((0, 0), (0, D - Wf2.shape[1])))
    bf2p = jnp.pad(bf2, (0, D - bf2.shape[0])).reshape(1, D)
    b1_1r, b3_1r = b1_1.reshape(1, D), b3_1.reshape(1, D)
    b1_2r, b3_2r = b1_2.reshape(1, D), b3_2.reshape(1, D)
    bf1r = bf1.reshape(1, D)

    src = jnp.concatenate(
        [edge_index[0], jnp.zeros((EPAD - N_EDGES,), jnp.int32)]
    ).reshape(NW, ENB, EIB, ECHUNK)
    dst = jnp.concatenate(
        [edge_index[1], jnp.full((EPAD - N_EDGES,), N_NODES, jnp.int32)]
    ).reshape(NW, ENB, EIB, ECHUNK)

    pool_src = jnp.concatenate(
        [jnp.arange(N_NODES, dtype=jnp.int32),
         jnp.zeros((PPAD - N_NODES,), jnp.int32)]
    ).reshape(NW, 1, PIB, PCHUNK)
    pool_dst = jnp.concatenate(
        [batch, jnp.full((PPAD - N_NODES,), NUM_GRAPHS, jnp.int32)]
    ).reshape(NW, 1, PIB, PCHUNK)

    z_x = jnp.zeros((NP, DX), f32)
    z_h = jnp.zeros((NP, D), f32)
    z_pool = jnp.zeros((NPOOL, DAUG), f32)

    parts0 = _sc_scatter_x(x_aug, src, dst, z_x)
    h1, deg = _tc_layer0(parts0, x_aug, w1aug, w2aug, w3aug)
    parts1 = _sc_scatter_h(h1, src, dst, z_h)
    h2 = _tc_layer(parts1, h1, deg, W1_1, b1_1r, W2_1, W3_1, b3_1r, False)
    parts2 = _sc_scatter_h(h2, src, dst, z_h)
    h3_aug = _tc_layer(parts2, h2, deg, W1_2, b1_2r, W2_2, W3_2, b3_2r, True)
    pool_parts = _sc_pool(h3_aug, pool_src, pool_dst, z_pool)
    pred_pad = _tc_head(pool_parts, Wf1, bf1r, wf2, bf2p)
    return pred_pad[:, :Wf2.shape[1]]
